# Initial kernel scaffold; baseline (speedup 1.0000x reference)
import jax
import jax.numpy as jnp
from jax.experimental import pallas as pl


def kernel(x_idx_sg1, x_float_sg1, x_idx_sg2, x_float_sg2, edge_index_sg1, edge_index_sg2, emb0, emb1, W0, b0, W1, b1, W2, b2, Wg1a, bg1a, Wg1b, bg1b, Wg2a, bg2a, Wg2b, bg2b, Wk, smoothing, M):
    raise NotImplementedError("write your pallas kernel here")


# speedup vs baseline: 1.0001x; 1.0001x over previous
import jax
import jax.numpy as jnp
from jax.experimental import pallas as pl

PAST = 12
FUTURE = 4
OUT_PRE = 27


def _gcn(x, edge_index, W, b, n):
    src = edge_index[0]
    dst = edge_index[1]
    loop = jnp.arange(n, dtype=src.dtype)
    s = jnp.concatenate([src, loop])
    d = jnp.concatenate([dst, loop])
    h = x @ W
    deg = jnp.zeros((n,), x.dtype).at[d].add(1.0)
    dinv = 1.0 / jnp.sqrt(deg)
    norm = dinv[s] * dinv[d]
    agg = jnp.zeros((n, W.shape[1]), x.dtype).at[d].add(jnp.take(h, s, axis=0) * norm[:, None])
    return agg + b


def _pre(idx, xf, e0, e1, W0, b0, W1, b1, W2, b2):
    emb = jnp.concatenate([jnp.take(e0, idx[:, 0], axis=0), jnp.take(e1, idx[:, 1], axis=0)], axis=-1)
    o = jnp.concatenate([emb, xf], axis=-1)
    h = jax.nn.relu(o @ W0 + b0)
    h = jax.nn.relu(h @ W1 + b1)
    h = h @ W2 + b2
    return jnp.concatenate([h, xf[:, -1:]], axis=-1)


def _kernelfn(xk, yk, A, Wk, smoothing):
    xp = xk[:, :PAST, :]
    xf = xk[:, PAST:, :]
    theta = Wk @ Wk.T
    theta = (theta + theta.T) / 2.0
    diff = xp[:, None, :, :] - xf[:, :, None, :]
    w = jnp.einsum('bfpo,oq,bfpq->bfp', diff, theta, diff)
    w = -0.5 * w / (jax.nn.sigmoid(smoothing) * 0.01)
    A_tmp = A[PAST:, :PAST]
    alpha = jnp.where(A_tmp[None, :, :] == 0, -jnp.inf, w)
    alpha = jax.nn.softmax(alpha, axis=-1)
    return jnp.matmul(alpha, yk)


def kernel(x_idx_sg1, x_float_sg1, x_idx_sg2, x_float_sg2, edge_index_sg1, edge_index_sg2,
           emb0, emb1, W0, b0, W1, b1, W2, b2,
           Wg1a, bg1a, Wg1b, bg1b, Wg2a, bg2a, Wg2b, bg2b,
           Wk, smoothing, M):
    N1 = x_idx_sg1.shape[0]
    N2 = x_idx_sg2.shape[0]
    A = jax.nn.relu(M @ M.T)
    mask = jnp.tril(jnp.ones((PAST + FUTURE, PAST + FUTURE), dtype=A.dtype))
    A = jnp.where(mask == 0, -jnp.inf, A)
    A = jax.nn.softmax(A, axis=1)
    x = _pre(x_idx_sg1, x_float_sg1, emb0, emb1, W0, b0, W1, b1, W2, b2)
    x2 = _pre(x_idx_sg2, x_float_sg2, emb0, emb1, W0, b0, W1, b1, W2, b2).reshape(-1, FUTURE, OUT_PRE)
    h = jax.nn.relu(_gcn(x, edge_index_sg1, Wg1a, bg1a, N1))
    x1 = _gcn(h, edge_index_sg1, Wg1b, bg1b, N1).reshape(-1, PAST, OUT_PRE - 1)
    x_kernel = jnp.concatenate([x1, x2[:, :, :-1]], axis=-2)
    y_kernel = x.reshape(-1, PAST, OUT_PRE)[:, :, -1:]
    yh = _kernelfn(x_kernel, y_kernel, A, Wk, smoothing)
    x2b = jnp.concatenate([x2[:, :, :-1], yh], axis=-1)
    g = jax.nn.relu(_gcn(x2b.reshape(-1, OUT_PRE), edge_index_sg2, Wg2a, bg2a, N2))
    out = _gcn(g, edge_index_sg2, Wg2b, bg2b, N2).reshape(-1, FUTURE)
    xc = jnp.concatenate([x.reshape(-1, PAST, OUT_PRE), x2b], axis=-2)
    d2 = jnp.sum((xc[:, :, None, :] - xc[:, None, :, :]) ** 2, axis=-1)
    dist = jnp.sqrt(jnp.maximum(d2, 1e-12))
    return out, dist, A


# speedup vs baseline: 24.4805x; 24.4791x over previous
import functools

import jax
import jax.numpy as jnp
from jax import lax
from jax.experimental import pallas as pl
from jax.experimental.pallas import tpu as pltpu
from jax.experimental.pallas import tpu_sc as plsc

PAST = 12
FUTURE = 4
OUT_PRE = 27

_NC = 2
_NS = 16


def _sc_mesh():
    return plsc.VectorSubcoreMesh(core_axis_name="c", subcore_axis_name="s")


def _make_deg_kernel(n_nodes, n_edges):
    e_per = n_edges // (_NC * _NS)
    n_vec = e_per // 16

    @functools.partial(
        pl.kernel,
        mesh=_sc_mesh(),
        compiler_params=pltpu.CompilerParams(needs_layout_passes=False, use_tc_tiling_on_sc=False),
        out_type=jax.ShapeDtypeStruct((_NC * _NS, n_nodes), jnp.float32),
        scratch_types=[
            pltpu.VMEM((n_nodes,), jnp.float32),
            pltpu.VMEM((e_per,), jnp.int32),
        ],
    )
    def k(d_hbm, z_hbm, out_hbm, acc_v, didx_v):
        c = lax.axis_index("c")
        s = lax.axis_index("s")
        wid = s * _NC + c
        base = wid * e_per
        pltpu.sync_copy(z_hbm.at[pl.ds(0, n_nodes)], acc_v)
        pltpu.sync_copy(d_hbm.at[pl.ds(base, e_per)], didx_v)
        ones = jnp.full((16,), 1.0, jnp.float32)

        def body(i, _):
            dv = didx_v[pl.ds(pl.multiple_of(i * 16, 16), 16)]
            plsc.addupdate_scatter(acc_v, [dv], ones)
            return ()

        lax.fori_loop(0, n_vec, body, (), unroll=4)
        pltpu.sync_copy(acc_v, out_hbm.at[wid])

    return k


def _make_scalar_scatter_kernel(n_nodes, n_edges):
    e_per = n_edges // (_NC * _NS)
    n_vec = e_per // 16

    @functools.partial(
        pl.kernel,
        mesh=_sc_mesh(),
        compiler_params=pltpu.CompilerParams(needs_layout_passes=False, use_tc_tiling_on_sc=False),
        out_type=jax.ShapeDtypeStruct((_NC * _NS, n_nodes), jnp.float32),
        scratch_types=[
            pltpu.VMEM((n_nodes,), jnp.float32),
            pltpu.VMEM((n_nodes,), jnp.float32),
            pltpu.VMEM((e_per,), jnp.int32),
            pltpu.VMEM((e_per,), jnp.int32),
        ],
    )
    def k(s_hbm, d_hbm, vals_hbm, z_hbm, out_hbm, acc_v, vals_v, sidx_v, didx_v):
        c = lax.axis_index("c")
        s = lax.axis_index("s")
        wid = s * _NC + c
        base = wid * e_per
        pltpu.sync_copy(z_hbm.at[pl.ds(0, n_nodes)], acc_v)
        pltpu.sync_copy(vals_hbm, vals_v)
        pltpu.sync_copy(s_hbm.at[pl.ds(base, e_per)], sidx_v)
        pltpu.sync_copy(d_hbm.at[pl.ds(base, e_per)], didx_v)

        def body(i, _):
            o = pl.multiple_of(i * 16, 16)
            sv = sidx_v[pl.ds(o, 16)]
            dv = didx_v[pl.ds(o, 16)]
            val = plsc.load_gather(vals_v, [sv])
            plsc.addupdate_scatter(acc_v, [dv], val)
            return ()

        lax.fori_loop(0, n_vec, body, (), unroll=4)
        pltpu.sync_copy(acc_v, out_hbm.at[wid])

    return k


def _make_row_scatter_kernel(n_nodes, n_edges, edge_split):
    D = 32
    CH = 2048 if n_nodes <= 16384 else 512
    n_workers = _NS * (2 if edge_split else 1)
    e_per = n_edges // n_workers
    n_chunks = e_per // CH
    rows_per_tile = n_nodes // _NS

    @functools.partial(
        pl.kernel,
        mesh=_sc_mesh(),
        compiler_params=pltpu.CompilerParams(needs_layout_passes=False, use_tc_tiling_on_sc=False),
        out_type=jax.ShapeDtypeStruct((_NC, n_nodes, D), jnp.float32),
        scratch_types=[
            pltpu.VMEM_SHARED((n_nodes, D), jnp.float32),
            pltpu.VMEM((CH,), jnp.int32),
            pltpu.VMEM((CH,), jnp.int32),
            pltpu.VMEM((CH, D), jnp.float32),
            pltpu.SemaphoreType.DMA,
        ],
    )
    def k(hpA_hbm, hpB_hbm, s_hbm, d_hbm, z2d_hbm, out_hbm,
          acc_sp, sidx_v, didx_v, rows_v, sem):
        c = lax.axis_index("c")
        t = lax.axis_index("s")
        pltpu.sync_copy(z2d_hbm.at[pl.ds(0, rows_per_tile)],
                        acc_sp.at[pl.ds(t * rows_per_tile, rows_per_tile)])
        plsc.subcore_barrier()

        def run(hp_hbm):
            if edge_split:
                e0 = (c * _NS + t) * e_per
            else:
                e0 = t * e_per

            def chunk(i, _):
                r = e0 + i * CH
                pltpu.sync_copy(s_hbm.at[pl.ds(r, CH)], sidx_v)
                pltpu.sync_copy(d_hbm.at[pl.ds(r, CH)], didx_v)
                pltpu.async_copy(hp_hbm.at[sidx_v], rows_v, sem).wait()
                pltpu.sync_copy(rows_v, acc_sp.at[didx_v], add=True)
                return ()

            lax.fori_loop(0, n_chunks, chunk, ())

        @pl.when(c == 0)
        def _():
            run(hpA_hbm)

        @pl.when(c == 1)
        def _():
            run(hpB_hbm)

        plsc.subcore_barrier()
        pltpu.sync_copy(acc_sp.at[pl.ds(t * rows_per_tile, rows_per_tile)],
                        out_hbm.at[c, pl.ds(t * rows_per_tile, rows_per_tile)])

    return k


def _pre(idx, xf, e0, e1, W0, b0, W1, b1, W2, b2):
    emb = jnp.concatenate([jnp.take(e0, idx[:, 0], axis=0), jnp.take(e1, idx[:, 1], axis=0)], axis=-1)
    o = jnp.concatenate([emb, xf], axis=-1)
    h = jax.nn.relu(o @ W0 + b0)
    h = jax.nn.relu(h @ W1 + b1)
    h = h @ W2 + b2
    return jnp.concatenate([h, xf[:, -1:]], axis=-1)


def _att(xk, yk, A, Wk, smoothing):
    xp = xk[:, :PAST, :]
    xf = xk[:, PAST:, :]
    theta = Wk @ Wk.T
    theta = (theta + theta.T) / 2.0
    diff = xp[:, None, :, :] - xf[:, :, None, :]
    w = jnp.einsum('bfpo,oq,bfpq->bfp', diff, theta, diff)
    w = -0.5 * w / (jax.nn.sigmoid(smoothing) * 0.01)
    A_tmp = A[PAST:, :PAST]
    alpha = jnp.where(A_tmp[None, :, :] == 0, -jnp.inf, w)
    alpha = jax.nn.softmax(alpha, axis=-1)
    return jnp.matmul(alpha, yk)


def kernel(x_idx_sg1, x_float_sg1, x_idx_sg2, x_float_sg2, edge_index_sg1, edge_index_sg2,
           emb0, emb1, W0, b0, W1, b1, W2, b2,
           Wg1a, bg1a, Wg1b, bg1b, Wg2a, bg2a, Wg2b, bg2b,
           Wk, smoothing, M):
    N1 = x_idx_sg1.shape[0]
    N2 = x_idx_sg2.shape[0]
    E1 = edge_index_sg1.shape[1]
    E2 = edge_index_sg2.shape[1]

    s1 = edge_index_sg1[0]
    d1 = edge_index_sg1[1]
    s2 = edge_index_sg2[0]
    d2 = edge_index_sg2[1]
    z_flat = jnp.zeros((N1,), jnp.float32)
    z_2d = jnp.zeros((N1 // _NS, 32), jnp.float32)

    deg1 = _make_deg_kernel(N1, E1)(d1, z_flat).sum(axis=0) + 1.0
    deg2 = _make_deg_kernel(N2, E2)(d2, z_flat).sum(axis=0) + 1.0
    dinv1 = jax.lax.rsqrt(deg1)
    dinv2 = jax.lax.rsqrt(deg2)

    A = jax.nn.relu(M @ M.T)
    mask = jnp.tril(jnp.ones((PAST + FUTURE, PAST + FUTURE), dtype=A.dtype))
    A = jnp.where(mask == 0, -jnp.inf, A)
    A = jax.nn.softmax(A, axis=1)

    x = _pre(x_idx_sg1, x_float_sg1, emb0, emb1, W0, b0, W1, b1, W2, b2)
    x2f = _pre(x_idx_sg2, x_float_sg2, emb0, emb1, W0, b0, W1, b1, W2, b2)
    x2 = x2f.reshape(-1, FUTURE, OUT_PRE)

    hp1 = (x @ Wg1a) * dinv1[:, None]
    S1 = _make_row_scatter_kernel(N1, E1, False)(
        hp1[:, :32], hp1[:, 32:], s1, d1, z_2d)
    S1c = jnp.concatenate([S1[0], S1[1]], axis=1)
    h = jax.nn.relu(dinv1[:, None] * (S1c + hp1) + bg1a)

    hp2 = (h @ Wg1b) * dinv1[:, None]
    hp2p = jnp.pad(hp2, ((0, 0), (0, 6)))
    S2 = _make_row_scatter_kernel(N1, E1, True)(
        hp2p, hp2p, s1, d1, z_2d)
    S2s = S2[0] + S2[1]
    x1 = (dinv1[:, None] * (S2s[:, :26] + hp2) + bg1b).reshape(-1, PAST, OUT_PRE - 1)

    x_kernel = jnp.concatenate([x1, x2[:, :, :-1]], axis=-2)
    y_kernel = x.reshape(-1, PAST, OUT_PRE)[:, :, -1:]
    yh = _att(x_kernel, y_kernel, A, Wk, smoothing)
    x2b = jnp.concatenate([x2[:, :, :-1], yh], axis=-1)
    x2bf = x2b.reshape(-1, OUT_PRE)

    hp3 = (x2bf @ Wg2a) * dinv2[:, None]
    S3 = _make_row_scatter_kernel(N2, E2, False)(
        hp3[:, :32], hp3[:, 32:], s2, d2, z_2d)
    S3c = jnp.concatenate([S3[0], S3[1]], axis=1)
    g = jax.nn.relu(dinv2[:, None] * (S3c + hp3) + bg2a)

    hp4 = ((g @ Wg2b)[:, 0] + 0.0) * dinv2
    S4 = _make_scalar_scatter_kernel(N2, E2)(s2, d2, hp4, z_flat).sum(axis=0)
    out = (dinv2 * (S4 + hp4) + bg2b[0]).reshape(-1, FUTURE)

    xc = jnp.concatenate([x.reshape(-1, PAST, OUT_PRE), x2b], axis=-2)
    d2_ = jnp.sum((xc[:, :, None, :] - xc[:, None, :, :]) ** 2, axis=-1)
    dist = jnp.sqrt(jnp.maximum(d2_, 1e-12))
    return out, dist, A


# speedup vs baseline: 26.2860x; 1.0738x over previous
import functools

import jax
import jax.numpy as jnp
from jax import lax
from jax.experimental import pallas as pl
from jax.experimental.pallas import tpu as pltpu
from jax.experimental.pallas import tpu_sc as plsc

PAST = 12
FUTURE = 4
OUT_PRE = 27

_NC = 2
_NS = 16

_SC_PARAMS = dict(
    compiler_params=pltpu.CompilerParams(
        needs_layout_passes=False, use_tc_tiling_on_sc=False),
    mesh=plsc.VectorSubcoreMesh(core_axis_name="c", subcore_axis_name="s"),
)


def _make_emb_gather_kernel(n_rows):
    CH = 256
    r_per = n_rows // (_NC * _NS)
    n_chunks = r_per // CH

    @functools.partial(
        pl.kernel,
        out_type=(jax.ShapeDtypeStruct((n_rows, 128), jnp.float32),
                  jax.ShapeDtypeStruct((n_rows, 128), jnp.float32)),
        scratch_types=[
            pltpu.VMEM((CH,), jnp.int32),
            pltpu.VMEM((CH,), jnp.int32),
            pltpu.VMEM((CH, 128), jnp.float32),
            pltpu.VMEM((CH, 128), jnp.float32),
            pltpu.SemaphoreType.DMA,
            pltpu.SemaphoreType.DMA,
        ],
        **_SC_PARAMS,
    )
    def k(T0_hbm, T1_hbm, i0_hbm, i1_hbm, g0_hbm, g1_hbm,
          i0_v, i1_v, r0_v, r1_v, sem0, sem1):
        c = lax.axis_index("c")
        t = lax.axis_index("s")
        base = (t * _NC + c) * r_per

        def chunk(i, _):
            r = base + i * CH
            pltpu.sync_copy(i0_hbm.at[pl.ds(r, CH)], i0_v)
            pltpu.sync_copy(i1_hbm.at[pl.ds(r, CH)], i1_v)
            cp0 = pltpu.async_copy(T0_hbm.at[i0_v], r0_v, sem0)
            cp1 = pltpu.async_copy(T1_hbm.at[i1_v], r1_v, sem1)
            cp0.wait()
            pltpu.sync_copy(r0_v, g0_hbm.at[pl.ds(r, CH)])
            cp1.wait()
            pltpu.sync_copy(r1_v, g1_hbm.at[pl.ds(r, CH)])
            return ()

        lax.fori_loop(0, n_chunks, chunk, ())

    return k


def _make_deg_kernel(n_nodes, n_edges):
    e_per = n_edges // (_NC * _NS)
    n_vec = e_per // 16

    @functools.partial(
        pl.kernel,
        out_type=jax.ShapeDtypeStruct((_NC * _NS, n_nodes), jnp.float32),
        scratch_types=[
            pltpu.VMEM((n_nodes,), jnp.float32),
            pltpu.VMEM((e_per,), jnp.int32),
        ],
        **_SC_PARAMS,
    )
    def k(d_hbm, z_hbm, out_hbm, acc_v, didx_v):
        c = lax.axis_index("c")
        s = lax.axis_index("s")
        wid = s * _NC + c
        base = wid * e_per
        pltpu.sync_copy(z_hbm.at[pl.ds(0, n_nodes)], acc_v)
        pltpu.sync_copy(d_hbm.at[pl.ds(base, e_per)], didx_v)
        ones = jnp.full((16,), 1.0, jnp.float32)

        def body(i, _):
            dv = didx_v[pl.ds(pl.multiple_of(i * 16, 16), 16)]
            plsc.addupdate_scatter(acc_v, [dv], ones)
            return ()

        lax.fori_loop(0, n_vec, body, (), unroll=4)
        pltpu.sync_copy(acc_v, out_hbm.at[wid])

    return k


def _make_scalar_scatter_kernel(n_nodes, n_edges):
    e_per = n_edges // (_NC * _NS)
    n_vec = e_per // 16

    @functools.partial(
        pl.kernel,
        out_type=jax.ShapeDtypeStruct((_NC * _NS, n_nodes), jnp.float32),
        scratch_types=[
            pltpu.VMEM((n_nodes,), jnp.float32),
            pltpu.VMEM((n_nodes,), jnp.float32),
            pltpu.VMEM((e_per,), jnp.int32),
            pltpu.VMEM((e_per,), jnp.int32),
        ],
        **_SC_PARAMS,
    )
    def k(s_hbm, d_hbm, vals_hbm, z_hbm, out_hbm, acc_v, vals_v, sidx_v, didx_v):
        c = lax.axis_index("c")
        s = lax.axis_index("s")
        wid = s * _NC + c
        base = wid * e_per
        pltpu.sync_copy(z_hbm.at[pl.ds(0, n_nodes)], acc_v)
        pltpu.sync_copy(vals_hbm, vals_v)
        pltpu.sync_copy(s_hbm.at[pl.ds(base, e_per)], sidx_v)
        pltpu.sync_copy(d_hbm.at[pl.ds(base, e_per)], didx_v)

        def body(i, _):
            o = pl.multiple_of(i * 16, 16)
            sv = sidx_v[pl.ds(o, 16)]
            dv = didx_v[pl.ds(o, 16)]
            val = plsc.load_gather(vals_v, [sv])
            plsc.addupdate_scatter(acc_v, [dv], val)
            return ()

        lax.fori_loop(0, n_vec, body, (), unroll=4)
        pltpu.sync_copy(acc_v, out_hbm.at[wid])

    return k


def _make_row_scatter_kernel(n_nodes, n_edges, edge_split):
    D = 32
    CH = 2048 if n_nodes <= 16384 else 512
    n_workers = _NS * (2 if edge_split else 1)
    e_per = n_edges // n_workers
    n_chunks = e_per // CH
    rows_per_tile = n_nodes // _NS

    @functools.partial(
        pl.kernel,
        out_type=jax.ShapeDtypeStruct((_NC, n_nodes, D), jnp.float32),
        scratch_types=[
            pltpu.VMEM_SHARED((n_nodes, D), jnp.float32),
            pltpu.VMEM((CH,), jnp.int32),
            pltpu.VMEM((CH,), jnp.int32),
            pltpu.VMEM((CH, D), jnp.float32),
            pltpu.SemaphoreType.DMA,
        ],
        **_SC_PARAMS,
    )
    def k(hpA_hbm, hpB_hbm, s_hbm, d_hbm, z2d_hbm, out_hbm,
          acc_sp, sidx_v, didx_v, rows_v, sem):
        c = lax.axis_index("c")
        t = lax.axis_index("s")
        pltpu.sync_copy(z2d_hbm.at[pl.ds(0, rows_per_tile)],
                        acc_sp.at[pl.ds(t * rows_per_tile, rows_per_tile)])
        plsc.subcore_barrier()

        def run(hp_hbm):
            if edge_split:
                e0 = (c * _NS + t) * e_per
            else:
                e0 = t * e_per

            def chunk(i, _):
                r = e0 + i * CH
                pltpu.sync_copy(s_hbm.at[pl.ds(r, CH)], sidx_v)
                pltpu.sync_copy(d_hbm.at[pl.ds(r, CH)], didx_v)
                pltpu.async_copy(hp_hbm.at[sidx_v], rows_v, sem).wait()
                pltpu.sync_copy(rows_v, acc_sp.at[didx_v], add=True)
                return ()

            lax.fori_loop(0, n_chunks, chunk, ())

        @pl.when(c == 0)
        def _():
            run(hpA_hbm)

        @pl.when(c == 1)
        def _():
            run(hpB_hbm)

        plsc.subcore_barrier()
        pltpu.sync_copy(acc_sp.at[pl.ds(t * rows_per_tile, rows_per_tile)],
                        out_hbm.at[c, pl.ds(t * rows_per_tile, rows_per_tile)])

    return k


def _dot(a, b):
    return jnp.dot(a, b, preferred_element_type=jnp.float32)


def _prep_tc(emb0, emb1, W0, b0, Wk, M):
    def body(emb0_r, emb1_r, W0_r, b0_r, Wk_r, M_r, A_r, th_r, T0_r, T1_r):
        Mv = M_r[...]
        Araw = jnp.maximum(_dot(Mv, Mv.T), 0.0)
        P = PAST + FUTURE
        row = lax.broadcasted_iota(jnp.int32, (P, P), 0)
        col = lax.broadcasted_iota(jnp.int32, (P, P), 1)
        Am = jnp.where(col <= row, Araw, -jnp.inf)
        m = jnp.max(Am, axis=1, keepdims=True)
        e = jnp.exp(Am - m)
        A_r[...] = e / jnp.sum(e, axis=1, keepdims=True)
        Wkv = Wk_r[...]
        th = _dot(Wkv, Wkv.T)
        th_r[...] = (th + th.T) / 2.0
        W0v = W0_r[...]
        T0_r[...] = _dot(emb0_r[...], W0v[0:16, :]) + b0_r[...]
        T1_r[...] = _dot(emb1_r[...], W0v[16:24, :])

    return pl.pallas_call(
        body,
        out_shape=(jax.ShapeDtypeStruct((16, 16), jnp.float32),
                   jax.ShapeDtypeStruct((26, 26), jnp.float32),
                   jax.ShapeDtypeStruct((512, 128), jnp.float32),
                   jax.ShapeDtypeStruct((512, 128), jnp.float32)),
    )(emb0, emb1, W0, b0.reshape(1, 128), Wk, M)


def _pre_tc(G0, G1, xf, W0r, W1, b1, W2, b2):
    N = G0.shape[0]
    BN = 2048
    grid = N // BN

    def body(g0_r, g1_r, xf_r, W0r_r, W1_r, b1_r, W2_r, b2_r, x_r):
        xfv = xf_r[...]
        h0 = jnp.maximum(g0_r[...] + g1_r[...] + _dot(xfv, W0r_r[...]), 0.0)
        h1 = jnp.maximum(_dot(h0, W1_r[...]) + b1_r[...], 0.0)
        h2 = _dot(h1, W2_r[...]) + b2_r[...]
        x_r[...] = jnp.concatenate([h2, xfv[:, 2:3]], axis=1)

    return pl.pallas_call(
        body,
        grid=(grid,),
        in_specs=[
            pl.BlockSpec((BN, 128), lambda i: (i, 0)),
            pl.BlockSpec((BN, 128), lambda i: (i, 0)),
            pl.BlockSpec((BN, 3), lambda i: (i, 0)),
            pl.BlockSpec((3, 128), lambda i: (0, 0)),
            pl.BlockSpec((128, 128), lambda i: (0, 0)),
            pl.BlockSpec((1, 128), lambda i: (0, 0)),
            pl.BlockSpec((128, 26), lambda i: (0, 0)),
            pl.BlockSpec((1, 26), lambda i: (0, 0)),
        ],
        out_specs=pl.BlockSpec((BN, OUT_PRE), lambda i: (i, 0)),
        out_shape=jax.ShapeDtypeStruct((N, OUT_PRE), jnp.float32),
    )(G0, G1, xf, W0r, W1, b1.reshape(1, 128), W2, b2.reshape(1, 26))


def _hp_tc(x_all, degp, W, n_nodes):
    BN = 2048
    grid = n_nodes // BN

    def body(x_r, degp_r, W_r, hpA_r, hpB_r, dinv_r):
        deg = jnp.sum(degp_r[...], axis=0, keepdims=True) + 1.0
        dinv = lax.rsqrt(deg).T
        hp = _dot(x_r[...], W_r[...]) * dinv
        hpA_r[...] = hp[:, :32]
        hpB_r[...] = hp[:, 32:]
        dinv_r[...] = dinv

    return pl.pallas_call(
        body,
        grid=(grid,),
        in_specs=[
            pl.BlockSpec((BN, OUT_PRE), lambda i: (i, 0)),
            pl.BlockSpec((32, BN), lambda i: (0, i)),
            pl.BlockSpec((OUT_PRE, 64), lambda i: (0, 0)),
        ],
        out_specs=(pl.BlockSpec((BN, 32), lambda i: (i, 0)),
                   pl.BlockSpec((BN, 32), lambda i: (i, 0)),
                   pl.BlockSpec((BN, 1), lambda i: (i, 0))),
        out_shape=(jax.ShapeDtypeStruct((n_nodes, 32), jnp.float32),
                   jax.ShapeDtypeStruct((n_nodes, 32), jnp.float32),
                   jax.ShapeDtypeStruct((n_nodes, 1), jnp.float32)),
    )(x_all, degp, W)


def _mid_tc(S, hpA, hpB, dinv, b_in, W_next, n_nodes):
    BN = 2048
    grid = n_nodes // BN
    dn = W_next.shape[1]

    def body(S_r, hpA_r, hpB_r, dinv_r, b_r, Wn_r, out_r):
        Sv = S_r[...]
        hp = jnp.concatenate([hpA_r[...], hpB_r[...]], axis=1)
        S64 = jnp.concatenate([Sv[0], Sv[1]], axis=1)
        dv = dinv_r[...]
        h = jnp.maximum(dv * (S64 + hp) + b_r[...], 0.0)
        hp2 = _dot(h, Wn_r[...]) * dv
        out_r[...] = jnp.concatenate(
            [hp2, jnp.zeros((hp2.shape[0], 32 - dn), jnp.float32)], axis=1)

    return pl.pallas_call(
        body,
        grid=(grid,),
        in_specs=[
            pl.BlockSpec((2, BN, 32), lambda i: (0, i, 0)),
            pl.BlockSpec((BN, 32), lambda i: (i, 0)),
            pl.BlockSpec((BN, 32), lambda i: (i, 0)),
            pl.BlockSpec((BN, 1), lambda i: (i, 0)),
            pl.BlockSpec((1, 64), lambda i: (0, 0)),
            pl.BlockSpec((64, dn), lambda i: (0, 0)),
        ],
        out_specs=pl.BlockSpec((BN, 32), lambda i: (i, 0)),
        out_shape=jax.ShapeDtypeStruct((n_nodes, 32), jnp.float32),
    )(S, hpA, hpB, dinv, b_in.reshape(1, 64), W_next)


def _att_tc(S2, hp2p, dinv1, bg1b, x_all, A_tmp, theta, smoothing,
            deg2p, Wg2a, N1, N2):
    BB = 256
    R1 = BB * PAST
    R2 = BB * FUTURE
    grid = N2 // R2
    off2 = N1 // R2

    def body(S2_r, hp2p_r, dinv1_r, b1b_r, xg1_r, xg2_r, At_r, th_r, sm_r,
             degp_r, Wg2a_r, x2b_r, dinv2_r, hp3A_r, hp3B_r):
        Sv = S2_r[...]
        x1f = (dinv1_r[...] * (Sv[0] + Sv[1] + hp2p_r[...]))[:, :26] + b1b_r[...]
        xp = x1f.reshape(BB, PAST, 26)
        xg2 = xg2_r[...]
        xf26f = xg2[:, :26]
        xf26 = xf26f.reshape(BB, FUTURE, 26)
        y = xg1_r[...][:, 26].reshape(BB, PAST)

        th = th_r[...]
        Zp = _dot(x1f, th)
        qp = jnp.sum(Zp * x1f, axis=1).reshape(BB, PAST)
        Zf = _dot(xf26f, th)
        qf = jnp.sum(Zf * xf26f, axis=1).reshape(BB, FUTURE)
        Zf3 = Zf.reshape(BB, FUTURE, 26)
        C = jnp.sum(Zf3[:, :, None, :] * xp[:, None, :, :], axis=-1)

        sig = 1.0 / (1.0 + jnp.exp(-sm_r[0, 0]))
        w = (qp[:, None, :] + qf[:, :, None] - 2.0 * C) * (-0.5 / (0.01 * sig))
        At = At_r[...][None, :, :]
        w = jnp.where(At == 0.0, -jnp.inf, w)
        m = jnp.max(w, axis=2, keepdims=True)
        e = jnp.exp(w - m)
        alpha = e / jnp.sum(e, axis=2, keepdims=True)
        yh = jnp.sum(alpha * y[:, None, :], axis=2)

        x2b = jnp.concatenate(
            [xf26, yh[:, :, None]], axis=2).reshape(R2, OUT_PRE)
        x2b_r[...] = x2b

        deg = jnp.sum(degp_r[...], axis=0, keepdims=True) + 1.0
        dinv2 = lax.rsqrt(deg).T
        dinv2_r[...] = dinv2
        hp3 = _dot(x2b, Wg2a_r[...]) * dinv2
        hp3A_r[...] = hp3[:, :32]
        hp3B_r[...] = hp3[:, 32:]

    return pl.pallas_call(
        body,
        grid=(grid,),
        in_specs=[
            pl.BlockSpec((2, R1, 32), lambda i: (0, i, 0)),
            pl.BlockSpec((R1, 32), lambda i: (i, 0)),
            pl.BlockSpec((R1, 1), lambda i: (i, 0)),
            pl.BlockSpec((1, 26), lambda i: (0, 0)),
            pl.BlockSpec((R1, OUT_PRE), lambda i: (i, 0)),
            pl.BlockSpec((R2, OUT_PRE), lambda i, _o=off2: (_o + i, 0)),
            pl.BlockSpec((FUTURE, PAST), lambda i: (0, 0)),
            pl.BlockSpec((26, 26), lambda i: (0, 0)),
            pl.BlockSpec((1, 1), lambda i: (0, 0)),
            pl.BlockSpec((32, R2), lambda i: (0, i)),
            pl.BlockSpec((OUT_PRE, 64), lambda i: (0, 0)),
        ],
        out_specs=(pl.BlockSpec((R2, OUT_PRE), lambda i: (i, 0)),
                   pl.BlockSpec((R2, 1), lambda i: (i, 0)),
                   pl.BlockSpec((R2, 32), lambda i: (i, 0)),
                   pl.BlockSpec((R2, 32), lambda i: (i, 0))),
        out_shape=(jax.ShapeDtypeStruct((N2, OUT_PRE), jnp.float32),
                   jax.ShapeDtypeStruct((N2, 1), jnp.float32),
                   jax.ShapeDtypeStruct((N2, 32), jnp.float32),
                   jax.ShapeDtypeStruct((N2, 32), jnp.float32)),
    )(S2, hp2p, dinv1, bg1b.reshape(1, 26), x_all, x_all, A_tmp, theta,
      smoothing.reshape(1, 1), deg2p, Wg2a)


def _mid2_tc(S3, hp3A, hp3B, dinv2, bg2a, Wg2b, n_nodes):
    BN = 2048
    grid = n_nodes // BN

    def body(S_r, hpA_r, hpB_r, dinv_r, b_r, W_r, out_r):
        Sv = S_r[...]
        hp = jnp.concatenate([hpA_r[...], hpB_r[...]], axis=1)
        S64 = jnp.concatenate([Sv[0], Sv[1]], axis=1)
        dv = dinv_r[...]
        g = jnp.maximum(dv * (S64 + hp) + b_r[...], 0.0)
        out_r[...] = _dot(g, W_r[...]) * dv

    return pl.pallas_call(
        body,
        grid=(grid,),
        in_specs=[
            pl.BlockSpec((2, BN, 32), lambda i: (0, i, 0)),
            pl.BlockSpec((BN, 32), lambda i: (i, 0)),
            pl.BlockSpec((BN, 32), lambda i: (i, 0)),
            pl.BlockSpec((BN, 1), lambda i: (i, 0)),
            pl.BlockSpec((1, 64), lambda i: (0, 0)),
            pl.BlockSpec((64, 1), lambda i: (0, 0)),
        ],
        out_specs=pl.BlockSpec((BN, 1), lambda i: (i, 0)),
        out_shape=jax.ShapeDtypeStruct((n_nodes, 1), jnp.float32),
    )(S3, hp3A, hp3B, dinv2, bg2a.reshape(1, 64), Wg2b)


def _final_tc(S4p, hp4, dinv2, bg2b, x_all, x2b, N1, N2):
    BB = 256
    R1 = BB * PAST
    R2 = BB * FUTURE
    grid = N2 // R2
    off2 = N1 // R1
    B = N2 // FUTURE
    P = PAST + FUTURE

    def body(S4p_r, hp4_r, dinv2_r, b_r, xg1_r, x2b_r, out_r, dist_r):
        s4 = jnp.sum(S4p_r[...], axis=0).reshape(R2, 1)
        o = dinv2_r[...] * (s4 + hp4_r[...]) + b_r[...]
        out_r[...] = o.reshape(BB, FUTURE)

        xc = jnp.concatenate([xg1_r[...].reshape(BB, PAST, OUT_PRE),
                              x2b_r[...].reshape(BB, FUTURE, OUT_PRE)], axis=1)
        q = jnp.sum(xc * xc, axis=2)
        for i in range(P):
            gi = jnp.sum(xc * xc[:, i:i + 1, :], axis=2)
            d2 = q[:, i:i + 1] + q - 2.0 * gi
            dist_r[:, i, :] = jnp.sqrt(jnp.maximum(d2, 1e-12))

    return pl.pallas_call(
        body,
        grid=(grid,),
        in_specs=[
            pl.BlockSpec((32, R2), lambda i: (0, i)),
            pl.BlockSpec((R2, 1), lambda i: (i, 0)),
            pl.BlockSpec((R2, 1), lambda i: (i, 0)),
            pl.BlockSpec((1, 1), lambda i: (0, 0)),
            pl.BlockSpec((R1, OUT_PRE), lambda i: (i, 0)),
            pl.BlockSpec((R2, OUT_PRE), lambda i: (i, 0)),
        ],
        out_specs=(pl.BlockSpec((BB, FUTURE), lambda i: (i, 0)),
                   pl.BlockSpec((BB, P, P), lambda i: (i, 0, 0))),
        out_shape=(jax.ShapeDtypeStruct((B, FUTURE), jnp.float32),
                   jax.ShapeDtypeStruct((B, P, P), jnp.float32)),
    )(S4p, hp4, dinv2, bg2b.reshape(1, 1), x_all, x2b)


def kernel(x_idx_sg1, x_float_sg1, x_idx_sg2, x_float_sg2, edge_index_sg1, edge_index_sg2,
           emb0, emb1, W0, b0, W1, b1, W2, b2,
           Wg1a, bg1a, Wg1b, bg1b, Wg2a, bg2a, Wg2b, bg2b,
           Wk, smoothing, M):
    N1 = x_idx_sg1.shape[0]
    N2 = x_idx_sg2.shape[0]
    E1 = edge_index_sg1.shape[1]
    E2 = edge_index_sg2.shape[1]

    s1 = edge_index_sg1[0]
    d1 = edge_index_sg1[1]
    s2 = edge_index_sg2[0]
    d2 = edge_index_sg2[1]
    z_flat = jnp.zeros((N1,), jnp.float32)
    z_2d = jnp.zeros((N1 // _NS, 32), jnp.float32)

    A, theta, T0, T1 = _prep_tc(emb0, emb1, W0, b0, Wk, M)

    deg1p = _make_deg_kernel(N1, E1)(d1, z_flat)
    deg2p = _make_deg_kernel(N2, E2)(d2, z_flat)

    xidx = jnp.concatenate([x_idx_sg1, x_idx_sg2], axis=0)
    xf = jnp.concatenate([x_float_sg1, x_float_sg2], axis=0)
    i0 = xidx[:, 0]
    i1 = xidx[:, 1]
    G0, G1 = _make_emb_gather_kernel(N1 + N2)(T0, T1, i0, i1)
    x_all = _pre_tc(G0, G1, xf, W0[24:27, :], W1, b1, W2, b2)

    hp1A, hp1B, dinv1 = _hp_tc(x_all[:N1], deg1p, Wg1a, N1)
    S1 = _make_row_scatter_kernel(N1, E1, False)(hp1A, hp1B, s1, d1, z_2d)

    hp2p = _mid_tc(S1, hp1A, hp1B, dinv1, bg1a, Wg1b, N1)
    S2 = _make_row_scatter_kernel(N1, E1, True)(hp2p, hp2p, s1, d1, z_2d)

    A_tmp = lax.slice(A, (PAST, 0), (PAST + FUTURE, PAST))
    x2b, dinv2, hp3A, hp3B = _att_tc(
        S2, hp2p, dinv1, bg1b, x_all, A_tmp, theta, smoothing,
        deg2p, Wg2a, N1, N2)

    S3 = _make_row_scatter_kernel(N2, E2, False)(hp3A, hp3B, s2, d2, z_2d)

    hp4 = _mid2_tc(S3, hp3A, hp3B, dinv2, bg2a, Wg2b, N2)
    S4p = _make_scalar_scatter_kernel(N2, E2)(s2, d2, hp4[:, 0], z_flat)

    out, dist = _final_tc(S4p, hp4, dinv2, bg2b, x_all, x2b, N1, N2)
    return out, dist, A


# speedup vs baseline: 31.2668x; 1.1895x over previous
import functools

import jax
import jax.numpy as jnp
from jax import lax
from jax.experimental import pallas as pl
from jax.experimental.pallas import tpu as pltpu
from jax.experimental.pallas import tpu_sc as plsc

PAST = 12
FUTURE = 4
OUT_PRE = 27

_NC = 2
_NS = 16

_SC_PARAMS = dict(
    compiler_params=pltpu.CompilerParams(
        needs_layout_passes=False, use_tc_tiling_on_sc=False),
    mesh=plsc.VectorSubcoreMesh(core_axis_name="c", subcore_axis_name="s"),
)


def _make_emb_gather_kernel(n_rows):
    CH = 256
    r_per = n_rows // (_NC * _NS)
    n_chunks = r_per // CH

    @functools.partial(
        pl.kernel,
        out_type=(jax.ShapeDtypeStruct((n_rows, 128), jnp.float32),
                  jax.ShapeDtypeStruct((n_rows, 128), jnp.float32)),
        scratch_types=[
            pltpu.VMEM((CH,), jnp.int32),
            pltpu.VMEM((CH,), jnp.int32),
            pltpu.VMEM((CH, 128), jnp.float32),
            pltpu.VMEM((CH, 128), jnp.float32),
            pltpu.SemaphoreType.DMA,
            pltpu.SemaphoreType.DMA,
        ],
        **_SC_PARAMS,
    )
    def k(T0_hbm, T1_hbm, i0_hbm, i1_hbm, g0_hbm, g1_hbm,
          i0_v, i1_v, r0_v, r1_v, sem0, sem1):
        c = lax.axis_index("c")
        t = lax.axis_index("s")
        base = (t * _NC + c) * r_per

        def chunk(i, _):
            r = base + i * CH
            pltpu.sync_copy(i0_hbm.at[pl.ds(r, CH)], i0_v)
            pltpu.sync_copy(i1_hbm.at[pl.ds(r, CH)], i1_v)
            cp0 = pltpu.async_copy(T0_hbm.at[i0_v], r0_v, sem0)
            cp1 = pltpu.async_copy(T1_hbm.at[i1_v], r1_v, sem1)
            cp0.wait()
            pltpu.sync_copy(r0_v, g0_hbm.at[pl.ds(r, CH)])
            cp1.wait()
            pltpu.sync_copy(r1_v, g1_hbm.at[pl.ds(r, CH)])
            return ()

        lax.fori_loop(0, n_chunks, chunk, ())

    return k


def _make_deg_kernel(n_nodes, n_edges):
    e_per = n_edges // (_NC * _NS)
    n_vec = e_per // 16

    @functools.partial(
        pl.kernel,
        out_type=jax.ShapeDtypeStruct((_NC * _NS, n_nodes), jnp.float32),
        scratch_types=[
            pltpu.VMEM((n_nodes,), jnp.float32),
            pltpu.VMEM((e_per,), jnp.int32),
        ],
        **_SC_PARAMS,
    )
    def k(d_hbm, z_hbm, out_hbm, acc_v, didx_v):
        c = lax.axis_index("c")
        s = lax.axis_index("s")
        wid = s * _NC + c
        base = wid * e_per
        pltpu.sync_copy(z_hbm.at[pl.ds(0, n_nodes)], acc_v)
        pltpu.sync_copy(d_hbm.at[pl.ds(base, e_per)], didx_v)
        ones = jnp.full((16,), 1.0, jnp.float32)

        def body(i, _):
            dv = didx_v[pl.ds(pl.multiple_of(i * 16, 16), 16)]
            plsc.addupdate_scatter(acc_v, [dv], ones)
            return ()

        lax.fori_loop(0, n_vec, body, (), unroll=4)
        pltpu.sync_copy(acc_v, out_hbm.at[wid])

    return k


def _make_scalar_scatter_kernel(n_nodes, n_edges):
    e_per = n_edges // (_NC * _NS)
    n_vec = e_per // 16

    @functools.partial(
        pl.kernel,
        out_type=jax.ShapeDtypeStruct((_NC * _NS, n_nodes), jnp.float32),
        scratch_types=[
            pltpu.VMEM((n_nodes,), jnp.float32),
            pltpu.VMEM((n_nodes,), jnp.float32),
            pltpu.VMEM((e_per,), jnp.int32),
            pltpu.VMEM((e_per,), jnp.int32),
        ],
        **_SC_PARAMS,
    )
    def k(s_hbm, d_hbm, vals_hbm, z_hbm, out_hbm, acc_v, vals_v, sidx_v, didx_v):
        c = lax.axis_index("c")
        s = lax.axis_index("s")
        wid = s * _NC + c
        base = wid * e_per
        pltpu.sync_copy(z_hbm.at[pl.ds(0, n_nodes)], acc_v)
        pltpu.sync_copy(vals_hbm, vals_v)
        pltpu.sync_copy(s_hbm.at[pl.ds(base, e_per)], sidx_v)
        pltpu.sync_copy(d_hbm.at[pl.ds(base, e_per)], didx_v)

        def body(i, _):
            o = pl.multiple_of(i * 16, 16)
            sv = sidx_v[pl.ds(o, 16)]
            dv = didx_v[pl.ds(o, 16)]
            val = plsc.load_gather(vals_v, [sv])
            plsc.addupdate_scatter(acc_v, [dv], val)
            return ()

        lax.fori_loop(0, n_vec, body, (), unroll=4)
        pltpu.sync_copy(acc_v, out_hbm.at[wid])

    return k


def _make_row_scatter_kernel(n_nodes, n_edges, edge_split):
    D = 32
    CH = 1024 if n_nodes <= 16384 else 384
    n_workers = _NS * (2 if edge_split else 1)
    e_per = n_edges // n_workers
    n_chunks = e_per // CH
    assert n_chunks % 2 == 0
    rows_per_tile = n_nodes // _NS

    @functools.partial(
        pl.kernel,
        out_type=jax.ShapeDtypeStruct((_NC, n_nodes, D), jnp.float32),
        scratch_types=[
            pltpu.VMEM_SHARED((n_nodes, D), jnp.float32),
            pltpu.VMEM((CH,), jnp.int32), pltpu.VMEM((CH,), jnp.int32),
            pltpu.VMEM((CH,), jnp.int32), pltpu.VMEM((CH,), jnp.int32),
            pltpu.VMEM((CH, D), jnp.float32), pltpu.VMEM((CH, D), jnp.float32),
            pltpu.SemaphoreType.DMA, pltpu.SemaphoreType.DMA,
            pltpu.SemaphoreType.DMA, pltpu.SemaphoreType.DMA,
            pltpu.SemaphoreType.DMA, pltpu.SemaphoreType.DMA,
        ],
        **_SC_PARAMS,
    )
    def k(hpA_hbm, hpB_hbm, s_hbm, d_hbm, z2d_hbm, out_hbm,
          acc_sp, sidx0, didx0, sidx1, didx1, rows0, rows1,
          semi0, semi1, semg0, semg1, sems0, sems1):
        c = lax.axis_index("c")
        t = lax.axis_index("s")
        pltpu.sync_copy(z2d_hbm.at[pl.ds(0, rows_per_tile)],
                        acc_sp.at[pl.ds(t * rows_per_tile, rows_per_tile)])
        plsc.subcore_barrier()

        def run(hp_hbm):
            if edge_split:
                e0 = (c * _NS + t) * e_per
            else:
                e0 = t * e_per

            def step(j, _):
                ra = e0 + (2 * j) * CH
                rb = ra + CH
                ias = pltpu.async_copy(s_hbm.at[pl.ds(ra, CH)], sidx0, semi0)
                iad = pltpu.async_copy(d_hbm.at[pl.ds(ra, CH)], didx0, semi0)
                ibs = pltpu.async_copy(s_hbm.at[pl.ds(rb, CH)], sidx1, semi1)
                ibd = pltpu.async_copy(d_hbm.at[pl.ds(rb, CH)], didx1, semi1)
                ias.wait()
                iad.wait()
                ga = pltpu.async_copy(hp_hbm.at[sidx0], rows0, semg0)
                ibs.wait()
                ibd.wait()
                gb = pltpu.async_copy(hp_hbm.at[sidx1], rows1, semg1)
                ga.wait()
                sa = pltpu.async_copy(rows0, acc_sp.at[didx0], sems0, add=True)
                gb.wait()
                sb = pltpu.async_copy(rows1, acc_sp.at[didx1], sems1, add=True)
                sa.wait()
                sb.wait()
                return ()

            lax.fori_loop(0, n_chunks // 2, step, ())

        @pl.when(c == 0)
        def _():
            run(hpA_hbm)

        @pl.when(c == 1)
        def _():
            run(hpB_hbm)

        plsc.subcore_barrier()
        pltpu.sync_copy(acc_sp.at[pl.ds(t * rows_per_tile, rows_per_tile)],
                        out_hbm.at[c, pl.ds(t * rows_per_tile, rows_per_tile)])

    return k


def _dot(a, b):
    return jnp.dot(a, b, preferred_element_type=jnp.float32)


def _prep_tc(emb0, emb1, W0, b0, Wk, M):
    def body(emb0_r, emb1_r, W0_r, b0_r, Wk_r, M_r, A_r, th_r, T0_r, T1_r):
        Mv = M_r[...]
        Araw = jnp.maximum(_dot(Mv, Mv.T), 0.0)
        P = PAST + FUTURE
        row = lax.broadcasted_iota(jnp.int32, (P, P), 0)
        col = lax.broadcasted_iota(jnp.int32, (P, P), 1)
        Am = jnp.where(col <= row, Araw, -jnp.inf)
        m = jnp.max(Am, axis=1, keepdims=True)
        e = jnp.exp(Am - m)
        A_r[...] = e / jnp.sum(e, axis=1, keepdims=True)
        Wkv = Wk_r[...]
        th = _dot(Wkv, Wkv.T)
        th_r[...] = (th + th.T) / 2.0
        W0v = W0_r[...]
        T0_r[...] = _dot(emb0_r[...], W0v[0:16, :]) + b0_r[...]
        T1_r[...] = _dot(emb1_r[...], W0v[16:24, :])

    return pl.pallas_call(
        body,
        out_shape=(jax.ShapeDtypeStruct((16, 16), jnp.float32),
                   jax.ShapeDtypeStruct((26, 26), jnp.float32),
                   jax.ShapeDtypeStruct((512, 128), jnp.float32),
                   jax.ShapeDtypeStruct((512, 128), jnp.float32)),
    )(emb0, emb1, W0, b0.reshape(1, 128), Wk, M)


def _pre_tc(G0, G1, xf, W0r, W1, b1, W2, b2):
    N = G0.shape[0]
    BN = 2048
    grid = N // BN

    def body(g0_r, g1_r, xf_r, W0r_r, W1_r, b1_r, W2_r, b2_r, x_r):
        xfv = xf_r[...]
        h0 = jnp.maximum(g0_r[...] + g1_r[...] + _dot(xfv, W0r_r[...]), 0.0)
        h1 = jnp.maximum(_dot(h0, W1_r[...]) + b1_r[...], 0.0)
        h2 = _dot(h1, W2_r[...]) + b2_r[...]
        x_r[...] = jnp.concatenate([h2, xfv[:, 2:3]], axis=1)

    return pl.pallas_call(
        body,
        grid=(grid,),
        in_specs=[
            pl.BlockSpec((BN, 128), lambda i: (i, 0)),
            pl.BlockSpec((BN, 128), lambda i: (i, 0)),
            pl.BlockSpec((BN, 3), lambda i: (i, 0)),
            pl.BlockSpec((3, 128), lambda i: (0, 0)),
            pl.BlockSpec((128, 128), lambda i: (0, 0)),
            pl.BlockSpec((1, 128), lambda i: (0, 0)),
            pl.BlockSpec((128, 26), lambda i: (0, 0)),
            pl.BlockSpec((1, 26), lambda i: (0, 0)),
        ],
        out_specs=pl.BlockSpec((BN, OUT_PRE), lambda i: (i, 0)),
        out_shape=jax.ShapeDtypeStruct((N, OUT_PRE), jnp.float32),
    )(G0, G1, xf, W0r, W1, b1.reshape(1, 128), W2, b2.reshape(1, 26))


def _hp_tc(x_all, degp, W, n_nodes):
    BN = 2048
    grid = n_nodes // BN

    def body(x_r, degp_r, W_r, hpA_r, hpB_r, dinv_r):
        deg = jnp.sum(degp_r[...], axis=0, keepdims=True) + 1.0
        dinv = lax.rsqrt(deg).T
        hp = _dot(x_r[...], W_r[...]) * dinv
        hpA_r[...] = hp[:, :32]
        hpB_r[...] = hp[:, 32:]
        dinv_r[...] = dinv

    return pl.pallas_call(
        body,
        grid=(grid,),
        in_specs=[
            pl.BlockSpec((BN, OUT_PRE), lambda i: (i, 0)),
            pl.BlockSpec((32, BN), lambda i: (0, i)),
            pl.BlockSpec((OUT_PRE, 64), lambda i: (0, 0)),
        ],
        out_specs=(pl.BlockSpec((BN, 32), lambda i: (i, 0)),
                   pl.BlockSpec((BN, 32), lambda i: (i, 0)),
                   pl.BlockSpec((BN, 1), lambda i: (i, 0))),
        out_shape=(jax.ShapeDtypeStruct((n_nodes, 32), jnp.float32),
                   jax.ShapeDtypeStruct((n_nodes, 32), jnp.float32),
                   jax.ShapeDtypeStruct((n_nodes, 1), jnp.float32)),
    )(x_all, degp, W)


def _mid_tc(S, hpA, hpB, dinv, b_in, W_next, n_nodes):
    BN = 2048
    grid = n_nodes // BN
    dn = W_next.shape[1]

    def body(S_r, hpA_r, hpB_r, dinv_r, b_r, Wn_r, out_r):
        Sv = S_r[...]
        hp = jnp.concatenate([hpA_r[...], hpB_r[...]], axis=1)
        S64 = jnp.concatenate([Sv[0], Sv[1]], axis=1)
        dv = dinv_r[...]
        h = jnp.maximum(dv * (S64 + hp) + b_r[...], 0.0)
        hp2 = _dot(h, Wn_r[...]) * dv
        out_r[...] = jnp.concatenate(
            [hp2, jnp.zeros((hp2.shape[0], 32 - dn), jnp.float32)], axis=1)

    return pl.pallas_call(
        body,
        grid=(grid,),
        in_specs=[
            pl.BlockSpec((2, BN, 32), lambda i: (0, i, 0)),
            pl.BlockSpec((BN, 32), lambda i: (i, 0)),
            pl.BlockSpec((BN, 32), lambda i: (i, 0)),
            pl.BlockSpec((BN, 1), lambda i: (i, 0)),
            pl.BlockSpec((1, 64), lambda i: (0, 0)),
            pl.BlockSpec((64, dn), lambda i: (0, 0)),
        ],
        out_specs=pl.BlockSpec((BN, 32), lambda i: (i, 0)),
        out_shape=jax.ShapeDtypeStruct((n_nodes, 32), jnp.float32),
    )(S, hpA, hpB, dinv, b_in.reshape(1, 64), W_next)


def _att_tc(S2, hp2p, dinv1, bg1b, x_all, A_tmp, theta, smoothing,
            deg2p, Wg2a, N1, N2):
    BB = 256
    R1 = BB * PAST
    R2 = BB * FUTURE
    grid = N2 // R2
    off2 = N1 // R2

    def body(S2_r, hp2p_r, dinv1_r, b1b_r, xg1_r, xg2_r, At_r, th_r, sm_r,
             degp_r, Wg2a_r, x2b_r, dinv2_r, hp3A_r, hp3B_r):
        Sv = S2_r[...]
        x1f = (dinv1_r[...] * (Sv[0] + Sv[1] + hp2p_r[...]))[:, :26] + b1b_r[...]
        xp = x1f.reshape(BB, PAST, 26)
        xg2 = xg2_r[...]
        xf26f = xg2[:, :26]
        xf26 = xf26f.reshape(BB, FUTURE, 26)
        y = xg1_r[...][:, 26].reshape(BB, PAST)

        th = th_r[...]
        ws = []
        for f in range(FUTURE):
            dif = (xp - xf26[:, f:f + 1, :]).reshape(R1, 26)
            wf = jnp.sum(_dot(dif, th) * dif, axis=1)
            ws.append(wf.reshape(BB, 1, PAST))
        w = jnp.concatenate(ws, axis=1)

        sig = 1.0 / (1.0 + jnp.exp(-sm_r[0, 0]))
        w = w * (-0.5 / (0.01 * sig))
        At = At_r[...][None, :, :]
        w = jnp.where(At == 0.0, -jnp.inf, w)
        m = jnp.max(w, axis=2, keepdims=True)
        e = jnp.exp(w - m)
        alpha = e / jnp.sum(e, axis=2, keepdims=True)
        yh = jnp.sum(alpha * y[:, None, :], axis=2)

        x2b = jnp.concatenate(
            [xf26, yh[:, :, None]], axis=2).reshape(R2, OUT_PRE)
        x2b_r[...] = x2b

        deg = jnp.sum(degp_r[...], axis=0, keepdims=True) + 1.0
        dinv2 = lax.rsqrt(deg).T
        dinv2_r[...] = dinv2
        hp3 = _dot(x2b, Wg2a_r[...]) * dinv2
        hp3A_r[...] = hp3[:, :32]
        hp3B_r[...] = hp3[:, 32:]

    return pl.pallas_call(
        body,
        grid=(grid,),
        in_specs=[
            pl.BlockSpec((2, R1, 32), lambda i: (0, i, 0)),
            pl.BlockSpec((R1, 32), lambda i: (i, 0)),
            pl.BlockSpec((R1, 1), lambda i: (i, 0)),
            pl.BlockSpec((1, 26), lambda i: (0, 0)),
            pl.BlockSpec((R1, OUT_PRE), lambda i: (i, 0)),
            pl.BlockSpec((R2, OUT_PRE), lambda i, _o=off2: (_o + i, 0)),
            pl.BlockSpec((FUTURE, PAST), lambda i: (0, 0)),
            pl.BlockSpec((26, 26), lambda i: (0, 0)),
            pl.BlockSpec((1, 1), lambda i: (0, 0)),
            pl.BlockSpec((32, R2), lambda i: (0, i)),
            pl.BlockSpec((OUT_PRE, 64), lambda i: (0, 0)),
        ],
        out_specs=(pl.BlockSpec((R2, OUT_PRE), lambda i: (i, 0)),
                   pl.BlockSpec((R2, 1), lambda i: (i, 0)),
                   pl.BlockSpec((R2, 32), lambda i: (i, 0)),
                   pl.BlockSpec((R2, 32), lambda i: (i, 0))),
        out_shape=(jax.ShapeDtypeStruct((N2, OUT_PRE), jnp.float32),
                   jax.ShapeDtypeStruct((N2, 1), jnp.float32),
                   jax.ShapeDtypeStruct((N2, 32), jnp.float32),
                   jax.ShapeDtypeStruct((N2, 32), jnp.float32)),
    )(S2, hp2p, dinv1, bg1b.reshape(1, 26), x_all, x_all, A_tmp, theta,
      smoothing.reshape(1, 1), deg2p, Wg2a)


def _mid2_tc(S3, hp3A, hp3B, dinv2, bg2a, Wg2b, n_nodes):
    BN = 2048
    grid = n_nodes // BN

    def body(S_r, hpA_r, hpB_r, dinv_r, b_r, W_r, out_r):
        Sv = S_r[...]
        hp = jnp.concatenate([hpA_r[...], hpB_r[...]], axis=1)
        S64 = jnp.concatenate([Sv[0], Sv[1]], axis=1)
        dv = dinv_r[...]
        g = jnp.maximum(dv * (S64 + hp) + b_r[...], 0.0)
        out_r[...] = _dot(g, W_r[...]) * dv

    return pl.pallas_call(
        body,
        grid=(grid,),
        in_specs=[
            pl.BlockSpec((2, BN, 32), lambda i: (0, i, 0)),
            pl.BlockSpec((BN, 32), lambda i: (i, 0)),
            pl.BlockSpec((BN, 32), lambda i: (i, 0)),
            pl.BlockSpec((BN, 1), lambda i: (i, 0)),
            pl.BlockSpec((1, 64), lambda i: (0, 0)),
            pl.BlockSpec((64, 1), lambda i: (0, 0)),
        ],
        out_specs=pl.BlockSpec((BN, 1), lambda i: (i, 0)),
        out_shape=jax.ShapeDtypeStruct((n_nodes, 1), jnp.float32),
    )(S3, hp3A, hp3B, dinv2, bg2a.reshape(1, 64), Wg2b)


def _final_tc(S4p, hp4, dinv2, bg2b, x_all, x2b, N1, N2):
    BB = 256
    R1 = BB * PAST
    R2 = BB * FUTURE
    grid = N2 // R2
    off2 = N1 // R1
    B = N2 // FUTURE
    P = PAST + FUTURE

    def body(S4p_r, hp4_r, dinv2_r, b_r, xg1_r, x2b_r, out_r, dist_r):
        s4 = jnp.sum(S4p_r[...], axis=0).reshape(R2, 1)
        o = dinv2_r[...] * (s4 + hp4_r[...]) + b_r[...]
        out_r[...] = o.reshape(BB, FUTURE)

        xc = jnp.concatenate([xg1_r[...].reshape(BB, PAST, OUT_PRE),
                              x2b_r[...].reshape(BB, FUTURE, OUT_PRE)], axis=1)
        for i in range(P):
            dif = xc - xc[:, i:i + 1, :]
            d2 = jnp.sum(dif * dif, axis=2)
            dist_r[:, i, :] = jnp.sqrt(jnp.maximum(d2, 1e-12))

    return pl.pallas_call(
        body,
        grid=(grid,),
        in_specs=[
            pl.BlockSpec((32, R2), lambda i: (0, i)),
            pl.BlockSpec((R2, 1), lambda i: (i, 0)),
            pl.BlockSpec((R2, 1), lambda i: (i, 0)),
            pl.BlockSpec((1, 1), lambda i: (0, 0)),
            pl.BlockSpec((R1, OUT_PRE), lambda i: (i, 0)),
            pl.BlockSpec((R2, OUT_PRE), lambda i: (i, 0)),
        ],
        out_specs=(pl.BlockSpec((BB, FUTURE), lambda i: (i, 0)),
                   pl.BlockSpec((BB, P, P), lambda i: (i, 0, 0))),
        out_shape=(jax.ShapeDtypeStruct((B, FUTURE), jnp.float32),
                   jax.ShapeDtypeStruct((B, P, P), jnp.float32)),
    )(S4p, hp4, dinv2, bg2b.reshape(1, 1), x_all, x2b)


def kernel(x_idx_sg1, x_float_sg1, x_idx_sg2, x_float_sg2, edge_index_sg1, edge_index_sg2,
           emb0, emb1, W0, b0, W1, b1, W2, b2,
           Wg1a, bg1a, Wg1b, bg1b, Wg2a, bg2a, Wg2b, bg2b,
           Wk, smoothing, M):
    N1 = x_idx_sg1.shape[0]
    N2 = x_idx_sg2.shape[0]
    E1 = edge_index_sg1.shape[1]
    E2 = edge_index_sg2.shape[1]

    s1 = edge_index_sg1[0]
    d1 = edge_index_sg1[1]
    s2 = edge_index_sg2[0]
    d2 = edge_index_sg2[1]
    z_flat = jnp.zeros((N1,), jnp.float32)
    z_2d = jnp.zeros((N1 // _NS, 32), jnp.float32)

    A, theta, T0, T1 = _prep_tc(emb0, emb1, W0, b0, Wk, M)

    deg1p = _make_deg_kernel(N1, E1)(d1, z_flat)
    deg2p = _make_deg_kernel(N2, E2)(d2, z_flat)

    xidx = jnp.concatenate([x_idx_sg1, x_idx_sg2], axis=0)
    xf = jnp.concatenate([x_float_sg1, x_float_sg2], axis=0)
    i0 = xidx[:, 0]
    i1 = xidx[:, 1]
    G0, G1 = _make_emb_gather_kernel(N1 + N2)(T0, T1, i0, i1)
    x_all = _pre_tc(G0, G1, xf, W0[24:27, :], W1, b1, W2, b2)

    hp1A, hp1B, dinv1 = _hp_tc(x_all[:N1], deg1p, Wg1a, N1)
    S1 = _make_row_scatter_kernel(N1, E1, False)(hp1A, hp1B, s1, d1, z_2d)

    hp2p = _mid_tc(S1, hp1A, hp1B, dinv1, bg1a, Wg1b, N1)
    S2 = _make_row_scatter_kernel(N1, E1, True)(hp2p, hp2p, s1, d1, z_2d)

    A_tmp = lax.slice(A, (PAST, 0), (PAST + FUTURE, PAST))
    x2b, dinv2, hp3A, hp3B = _att_tc(
        S2, hp2p, dinv1, bg1b, x_all, A_tmp, theta, smoothing,
        deg2p, Wg2a, N1, N2)

    S3 = _make_row_scatter_kernel(N2, E2, False)(hp3A, hp3B, s2, d2, z_2d)

    hp4 = _mid2_tc(S3, hp3A, hp3B, dinv2, bg2a, Wg2b, N2)
    S4p = _make_scalar_scatter_kernel(N2, E2)(s2, d2, hp4[:, 0], z_flat)

    out, dist = _final_tc(S4p, hp4, dinv2, bg2b, x_all, x2b, N1, N2)
    return out, dist, A


# speedup vs baseline: 32.2607x; 1.0318x over previous
import functools

import jax
import jax.numpy as jnp
from jax import lax
from jax.experimental import pallas as pl
from jax.experimental.pallas import tpu as pltpu
from jax.experimental.pallas import tpu_sc as plsc

PAST = 12
FUTURE = 4
OUT_PRE = 27

_NC = 2
_NS = 16

_SC_PARAMS = dict(
    compiler_params=pltpu.CompilerParams(
        needs_layout_passes=False, use_tc_tiling_on_sc=False),
    mesh=plsc.VectorSubcoreMesh(core_axis_name="c", subcore_axis_name="s"),
)


def _make_emb_gather_kernel(n_rows):
    CH = 256
    r_per = n_rows // (_NC * _NS)
    n_chunks = r_per // CH

    @functools.partial(
        pl.kernel,
        out_type=(jax.ShapeDtypeStruct((n_rows, 128), jnp.float32),
                  jax.ShapeDtypeStruct((n_rows, 128), jnp.float32)),
        scratch_types=[
            pltpu.VMEM((CH,), jnp.int32),
            pltpu.VMEM((CH,), jnp.int32),
            pltpu.VMEM((CH, 128), jnp.float32),
            pltpu.VMEM((CH, 128), jnp.float32),
            pltpu.SemaphoreType.DMA,
            pltpu.SemaphoreType.DMA,
        ],
        **_SC_PARAMS,
    )
    def k(T0_hbm, T1_hbm, i0_hbm, i1_hbm, g0_hbm, g1_hbm,
          i0_v, i1_v, r0_v, r1_v, sem0, sem1):
        c = lax.axis_index("c")
        t = lax.axis_index("s")
        base = (t * _NC + c) * r_per

        def chunk(i, _):
            r = base + i * CH
            pltpu.sync_copy(i0_hbm.at[pl.ds(r, CH)], i0_v)
            pltpu.sync_copy(i1_hbm.at[pl.ds(r, CH)], i1_v)
            cp0 = pltpu.async_copy(T0_hbm.at[i0_v], r0_v, sem0)
            cp1 = pltpu.async_copy(T1_hbm.at[i1_v], r1_v, sem1)
            cp0.wait()
            pltpu.sync_copy(r0_v, g0_hbm.at[pl.ds(r, CH)])
            cp1.wait()
            pltpu.sync_copy(r1_v, g1_hbm.at[pl.ds(r, CH)])
            return ()

        lax.fori_loop(0, n_chunks, chunk, ())

    return k


def _make_deg_kernel(n_nodes, n_edges):
    e_per = n_edges // (_NC * _NS)
    n_vec = e_per // 16

    @functools.partial(
        pl.kernel,
        out_type=jax.ShapeDtypeStruct((_NC * _NS, n_nodes), jnp.float32),
        scratch_types=[
            pltpu.VMEM((n_nodes,), jnp.float32),
            pltpu.VMEM((e_per,), jnp.int32),
        ],
        **_SC_PARAMS,
    )
    def k(d_hbm, z_hbm, out_hbm, acc_v, didx_v):
        c = lax.axis_index("c")
        s = lax.axis_index("s")
        wid = s * _NC + c
        base = wid * e_per
        pltpu.sync_copy(z_hbm.at[pl.ds(0, n_nodes)], acc_v)
        pltpu.sync_copy(d_hbm.at[pl.ds(base, e_per)], didx_v)
        ones = jnp.full((16,), 1.0, jnp.float32)

        def body(i, _):
            dv = didx_v[pl.ds(pl.multiple_of(i * 16, 16), 16)]
            plsc.addupdate_scatter(acc_v, [dv], ones)
            return ()

        lax.fori_loop(0, n_vec, body, (), unroll=4)
        pltpu.sync_copy(acc_v, out_hbm.at[wid])

    return k


def _make_scalar_scatter_kernel(n_nodes, n_edges):
    e_per = n_edges // (_NC * _NS)
    n_vec = e_per // 16

    @functools.partial(
        pl.kernel,
        out_type=jax.ShapeDtypeStruct((_NC * _NS, n_nodes), jnp.float32),
        scratch_types=[
            pltpu.VMEM((n_nodes,), jnp.float32),
            pltpu.VMEM((n_nodes,), jnp.float32),
            pltpu.VMEM((e_per,), jnp.int32),
            pltpu.VMEM((e_per,), jnp.int32),
        ],
        **_SC_PARAMS,
    )
    def k(s_hbm, d_hbm, vals_hbm, z_hbm, out_hbm, acc_v, vals_v, sidx_v, didx_v):
        c = lax.axis_index("c")
        s = lax.axis_index("s")
        wid = s * _NC + c
        base = wid * e_per
        pltpu.sync_copy(z_hbm.at[pl.ds(0, n_nodes)], acc_v)
        pltpu.sync_copy(vals_hbm, vals_v)
        pltpu.sync_copy(s_hbm.at[pl.ds(base, e_per)], sidx_v)
        pltpu.sync_copy(d_hbm.at[pl.ds(base, e_per)], didx_v)

        def body(i, _):
            o = pl.multiple_of(i * 16, 16)
            sv = sidx_v[pl.ds(o, 16)]
            dv = didx_v[pl.ds(o, 16)]
            val = plsc.load_gather(vals_v, [sv])
            plsc.addupdate_scatter(acc_v, [dv], val)
            return ()

        lax.fori_loop(0, n_vec, body, (), unroll=4)
        pltpu.sync_copy(acc_v, out_hbm.at[wid])

    return k


def _make_row_scatter_kernel(n_nodes, n_edges, edge_split):
    D = 32
    CH = 1024 if n_nodes <= 16384 else 384
    n_workers = _NS * (2 if edge_split else 1)
    e_per = n_edges // n_workers
    n_chunks = e_per // CH
    assert n_chunks % 4 == 0
    rows_per_tile = n_nodes // _NS

    @functools.partial(
        pl.kernel,
        out_type=jax.ShapeDtypeStruct((_NC, n_nodes, D), jnp.float32),
        scratch_types=[
            pltpu.VMEM_SHARED((n_nodes, D), jnp.float32),
            [pltpu.VMEM((CH,), jnp.int32) for _ in range(4)],
            [pltpu.VMEM((CH,), jnp.int32) for _ in range(4)],
            [pltpu.VMEM((CH, D), jnp.float32) for _ in range(2)],
            [pltpu.SemaphoreType.DMA for _ in range(4)],
            [pltpu.SemaphoreType.DMA for _ in range(2)],
            [pltpu.SemaphoreType.DMA for _ in range(2)],
        ],
        **_SC_PARAMS,
    )
    def k(hpA_hbm, hpB_hbm, s_hbm, d_hbm, z2d_hbm, out_hbm,
          acc_sp, sidx, didx, rows, semi, semg, sems):
        c = lax.axis_index("c")
        t = lax.axis_index("s")
        pltpu.sync_copy(z2d_hbm.at[pl.ds(0, rows_per_tile)],
                        acc_sp.at[pl.ds(t * rows_per_tile, rows_per_tile)])
        plsc.subcore_barrier()

        def run(hp_hbm):
            if edge_split:
                e0 = (c * _NS + t) * e_per
            else:
                e0 = t * e_per

            def step(j, _):
                r0 = e0 + (4 * j) * CH
                iw = []
                for q in range(4):
                    iw.append(pltpu.async_copy(
                        s_hbm.at[pl.ds(r0 + q * CH, CH)], sidx[q], semi[q]))
                    iw.append(pltpu.async_copy(
                        d_hbm.at[pl.ds(r0 + q * CH, CH)], didx[q], semi[q]))
                iw[0].wait()
                iw[1].wait()
                g0 = pltpu.async_copy(hp_hbm.at[sidx[0]], rows[0], semg[0])
                iw[2].wait()
                iw[3].wait()
                g1 = pltpu.async_copy(hp_hbm.at[sidx[1]], rows[1], semg[1])
                g0.wait()
                s0 = pltpu.async_copy(rows[0], acc_sp.at[didx[0]], sems[0], add=True)
                g1.wait()
                s1 = pltpu.async_copy(rows[1], acc_sp.at[didx[1]], sems[1], add=True)
                iw[4].wait()
                iw[5].wait()
                s0.wait()
                g2 = pltpu.async_copy(hp_hbm.at[sidx[2]], rows[0], semg[0])
                iw[6].wait()
                iw[7].wait()
                s1.wait()
                g3 = pltpu.async_copy(hp_hbm.at[sidx[3]], rows[1], semg[1])
                g2.wait()
                s2 = pltpu.async_copy(rows[0], acc_sp.at[didx[2]], sems[0], add=True)
                g3.wait()
                s3 = pltpu.async_copy(rows[1], acc_sp.at[didx[3]], sems[1], add=True)
                s2.wait()
                s3.wait()
                return ()

            lax.fori_loop(0, n_chunks // 4, step, ())

        @pl.when(c == 0)
        def _():
            run(hpA_hbm)

        @pl.when(c == 1)
        def _():
            run(hpB_hbm)

        plsc.subcore_barrier()
        pltpu.sync_copy(acc_sp.at[pl.ds(t * rows_per_tile, rows_per_tile)],
                        out_hbm.at[c, pl.ds(t * rows_per_tile, rows_per_tile)])

    return k


def _dot(a, b):
    return jnp.dot(a, b, preferred_element_type=jnp.float32)


def _prep_tc(emb0, emb1, W0, b0, Wk, M):
    def body(emb0_r, emb1_r, W0_r, b0_r, Wk_r, M_r, A_r, th_r, T0_r, T1_r):
        Mv = M_r[...]
        Araw = jnp.maximum(_dot(Mv, Mv.T), 0.0)
        P = PAST + FUTURE
        row = lax.broadcasted_iota(jnp.int32, (P, P), 0)
        col = lax.broadcasted_iota(jnp.int32, (P, P), 1)
        Am = jnp.where(col <= row, Araw, -jnp.inf)
        m = jnp.max(Am, axis=1, keepdims=True)
        e = jnp.exp(Am - m)
        A_r[...] = e / jnp.sum(e, axis=1, keepdims=True)
        Wkv = Wk_r[...]
        th = _dot(Wkv, Wkv.T)
        th_r[...] = (th + th.T) / 2.0
        W0v = W0_r[...]
        T0_r[...] = _dot(emb0_r[...], W0v[0:16, :]) + b0_r[...]
        T1_r[...] = _dot(emb1_r[...], W0v[16:24, :])

    return pl.pallas_call(
        body,
        out_shape=(jax.ShapeDtypeStruct((16, 16), jnp.float32),
                   jax.ShapeDtypeStruct((26, 26), jnp.float32),
                   jax.ShapeDtypeStruct((512, 128), jnp.float32),
                   jax.ShapeDtypeStruct((512, 128), jnp.float32)),
    )(emb0, emb1, W0, b0.reshape(1, 128), Wk, M)


def _pre_tc(G0, G1, xf, W0r, W1, b1, W2, b2):
    N = G0.shape[0]
    BN = 2048
    grid = N // BN

    def body(g0_r, g1_r, xf_r, W0r_r, W1_r, b1_r, W2_r, b2_r, x_r):
        xfv = xf_r[...]
        h0 = jnp.maximum(g0_r[...] + g1_r[...] + _dot(xfv, W0r_r[...]), 0.0)
        h1 = jnp.maximum(_dot(h0, W1_r[...]) + b1_r[...], 0.0)
        h2 = _dot(h1, W2_r[...]) + b2_r[...]
        x_r[...] = jnp.concatenate([h2, xfv[:, 2:3]], axis=1)

    return pl.pallas_call(
        body,
        grid=(grid,),
        in_specs=[
            pl.BlockSpec((BN, 128), lambda i: (i, 0)),
            pl.BlockSpec((BN, 128), lambda i: (i, 0)),
            pl.BlockSpec((BN, 3), lambda i: (i, 0)),
            pl.BlockSpec((3, 128), lambda i: (0, 0)),
            pl.BlockSpec((128, 128), lambda i: (0, 0)),
            pl.BlockSpec((1, 128), lambda i: (0, 0)),
            pl.BlockSpec((128, 26), lambda i: (0, 0)),
            pl.BlockSpec((1, 26), lambda i: (0, 0)),
        ],
        out_specs=pl.BlockSpec((BN, OUT_PRE), lambda i: (i, 0)),
        out_shape=jax.ShapeDtypeStruct((N, OUT_PRE), jnp.float32),
    )(G0, G1, xf, W0r, W1, b1.reshape(1, 128), W2, b2.reshape(1, 26))


def _hp_tc(x_all, degp, W, n_nodes):
    BN = 2048
    grid = n_nodes // BN

    def body(x_r, degp_r, W_r, hpA_r, hpB_r, dinv_r):
        deg = jnp.sum(degp_r[...], axis=0, keepdims=True) + 1.0
        dinv = lax.rsqrt(deg).T
        hp = _dot(x_r[...], W_r[...]) * dinv
        hpA_r[...] = hp[:, :32]
        hpB_r[...] = hp[:, 32:]
        dinv_r[...] = dinv

    return pl.pallas_call(
        body,
        grid=(grid,),
        in_specs=[
            pl.BlockSpec((BN, OUT_PRE), lambda i: (i, 0)),
            pl.BlockSpec((32, BN), lambda i: (0, i)),
            pl.BlockSpec((OUT_PRE, 64), lambda i: (0, 0)),
        ],
        out_specs=(pl.BlockSpec((BN, 32), lambda i: (i, 0)),
                   pl.BlockSpec((BN, 32), lambda i: (i, 0)),
                   pl.BlockSpec((BN, 1), lambda i: (i, 0))),
        out_shape=(jax.ShapeDtypeStruct((n_nodes, 32), jnp.float32),
                   jax.ShapeDtypeStruct((n_nodes, 32), jnp.float32),
                   jax.ShapeDtypeStruct((n_nodes, 1), jnp.float32)),
    )(x_all, degp, W)


def _mid_tc(S, hpA, hpB, dinv, b_in, W_next, n_nodes):
    BN = 2048
    grid = n_nodes // BN
    dn = W_next.shape[1]

    def body(S_r, hpA_r, hpB_r, dinv_r, b_r, Wn_r, out_r):
        Sv = S_r[...]
        hp = jnp.concatenate([hpA_r[...], hpB_r[...]], axis=1)
        S64 = jnp.concatenate([Sv[0], Sv[1]], axis=1)
        dv = dinv_r[...]
        h = jnp.maximum(dv * (S64 + hp) + b_r[...], 0.0)
        hp2 = _dot(h, Wn_r[...]) * dv
        out_r[...] = jnp.concatenate(
            [hp2, jnp.zeros((hp2.shape[0], 32 - dn), jnp.float32)], axis=1)

    return pl.pallas_call(
        body,
        grid=(grid,),
        in_specs=[
            pl.BlockSpec((2, BN, 32), lambda i: (0, i, 0)),
            pl.BlockSpec((BN, 32), lambda i: (i, 0)),
            pl.BlockSpec((BN, 32), lambda i: (i, 0)),
            pl.BlockSpec((BN, 1), lambda i: (i, 0)),
            pl.BlockSpec((1, 64), lambda i: (0, 0)),
            pl.BlockSpec((64, dn), lambda i: (0, 0)),
        ],
        out_specs=pl.BlockSpec((BN, 32), lambda i: (i, 0)),
        out_shape=jax.ShapeDtypeStruct((n_nodes, 32), jnp.float32),
    )(S, hpA, hpB, dinv, b_in.reshape(1, 64), W_next)


def _att_tc(S2, hp2p, dinv1, bg1b, x_all, A_tmp, theta, smoothing,
            deg2p, Wg2a, N1, N2):
    BB = 256
    R1 = BB * PAST
    R2 = BB * FUTURE
    grid = N2 // R2
    off2 = N1 // R2

    def body(S2_r, hp2p_r, dinv1_r, b1b_r, xg1_r, xg2_r, At_r, th_r, sm_r,
             degp_r, Wg2a_r, x2b_r, dinv2_r, hp3A_r, hp3B_r):
        Sv = S2_r[...]
        x1f = (dinv1_r[...] * (Sv[0] + Sv[1] + hp2p_r[...]))[:, :26] + b1b_r[...]
        xp = x1f.reshape(BB, PAST, 26)
        xg2 = xg2_r[...]
        xf26f = xg2[:, :26]
        xf26 = xf26f.reshape(BB, FUTURE, 26)
        y = xg1_r[...][:, 26].reshape(BB, PAST)

        th = th_r[...]
        ws = []
        for f in range(FUTURE):
            dif = (xp - xf26[:, f:f + 1, :]).reshape(R1, 26)
            wf = jnp.sum(_dot(dif, th) * dif, axis=1)
            ws.append(wf.reshape(BB, 1, PAST))
        w = jnp.concatenate(ws, axis=1)

        sig = 1.0 / (1.0 + jnp.exp(-sm_r[0, 0]))
        w = w * (-0.5 / (0.01 * sig))
        At = At_r[...][None, :, :]
        w = jnp.where(At == 0.0, -jnp.inf, w)
        m = jnp.max(w, axis=2, keepdims=True)
        e = jnp.exp(w - m)
        alpha = e / jnp.sum(e, axis=2, keepdims=True)
        yh = jnp.sum(alpha * y[:, None, :], axis=2)

        x2b = jnp.concatenate(
            [xf26, yh[:, :, None]], axis=2).reshape(R2, OUT_PRE)
        x2b_r[...] = x2b

        deg = jnp.sum(degp_r[...], axis=0, keepdims=True) + 1.0
        dinv2 = lax.rsqrt(deg).T
        dinv2_r[...] = dinv2
        hp3 = _dot(x2b, Wg2a_r[...]) * dinv2
        hp3A_r[...] = hp3[:, :32]
        hp3B_r[...] = hp3[:, 32:]

    return pl.pallas_call(
        body,
        grid=(grid,),
        in_specs=[
            pl.BlockSpec((2, R1, 32), lambda i: (0, i, 0)),
            pl.BlockSpec((R1, 32), lambda i: (i, 0)),
            pl.BlockSpec((R1, 1), lambda i: (i, 0)),
            pl.BlockSpec((1, 26), lambda i: (0, 0)),
            pl.BlockSpec((R1, OUT_PRE), lambda i: (i, 0)),
            pl.BlockSpec((R2, OUT_PRE), lambda i, _o=off2: (_o + i, 0)),
            pl.BlockSpec((FUTURE, PAST), lambda i: (0, 0)),
            pl.BlockSpec((26, 26), lambda i: (0, 0)),
            pl.BlockSpec((1, 1), lambda i: (0, 0)),
            pl.BlockSpec((32, R2), lambda i: (0, i)),
            pl.BlockSpec((OUT_PRE, 64), lambda i: (0, 0)),
        ],
        out_specs=(pl.BlockSpec((R2, OUT_PRE), lambda i: (i, 0)),
                   pl.BlockSpec((R2, 1), lambda i: (i, 0)),
                   pl.BlockSpec((R2, 32), lambda i: (i, 0)),
                   pl.BlockSpec((R2, 32), lambda i: (i, 0))),
        out_shape=(jax.ShapeDtypeStruct((N2, OUT_PRE), jnp.float32),
                   jax.ShapeDtypeStruct((N2, 1), jnp.float32),
                   jax.ShapeDtypeStruct((N2, 32), jnp.float32),
                   jax.ShapeDtypeStruct((N2, 32), jnp.float32)),
    )(S2, hp2p, dinv1, bg1b.reshape(1, 26), x_all, x_all, A_tmp, theta,
      smoothing.reshape(1, 1), deg2p, Wg2a)


def _mid2_tc(S3, hp3A, hp3B, dinv2, bg2a, Wg2b, n_nodes):
    BN = 2048
    grid = n_nodes // BN

    def body(S_r, hpA_r, hpB_r, dinv_r, b_r, W_r, out_r):
        Sv = S_r[...]
        hp = jnp.concatenate([hpA_r[...], hpB_r[...]], axis=1)
        S64 = jnp.concatenate([Sv[0], Sv[1]], axis=1)
        dv = dinv_r[...]
        g = jnp.maximum(dv * (S64 + hp) + b_r[...], 0.0)
        out_r[...] = _dot(g, W_r[...]) * dv

    return pl.pallas_call(
        body,
        grid=(grid,),
        in_specs=[
            pl.BlockSpec((2, BN, 32), lambda i: (0, i, 0)),
            pl.BlockSpec((BN, 32), lambda i: (i, 0)),
            pl.BlockSpec((BN, 32), lambda i: (i, 0)),
            pl.BlockSpec((BN, 1), lambda i: (i, 0)),
            pl.BlockSpec((1, 64), lambda i: (0, 0)),
            pl.BlockSpec((64, 1), lambda i: (0, 0)),
        ],
        out_specs=pl.BlockSpec((BN, 1), lambda i: (i, 0)),
        out_shape=jax.ShapeDtypeStruct((n_nodes, 1), jnp.float32),
    )(S3, hp3A, hp3B, dinv2, bg2a.reshape(1, 64), Wg2b)


def _final_tc(S4p, hp4, dinv2, bg2b, x_all, x2b, N1, N2):
    BB = 256
    R1 = BB * PAST
    R2 = BB * FUTURE
    grid = N2 // R2
    off2 = N1 // R1
    B = N2 // FUTURE
    P = PAST + FUTURE

    def body(S4p_r, hp4_r, dinv2_r, b_r, xg1_r, x2b_r, out_r, dist_r):
        s4 = jnp.sum(S4p_r[...], axis=0).reshape(R2, 1)
        o = dinv2_r[...] * (s4 + hp4_r[...]) + b_r[...]
        out_r[...] = o.reshape(BB, FUTURE)

        xc = jnp.concatenate([xg1_r[...].reshape(BB, PAST, OUT_PRE),
                              x2b_r[...].reshape(BB, FUTURE, OUT_PRE)], axis=1)
        for i in range(P):
            dif = xc - xc[:, i:i + 1, :]
            d2 = jnp.sum(dif * dif, axis=2)
            dist_r[:, i, :] = jnp.sqrt(jnp.maximum(d2, 1e-12))

    return pl.pallas_call(
        body,
        grid=(grid,),
        in_specs=[
            pl.BlockSpec((32, R2), lambda i: (0, i)),
            pl.BlockSpec((R2, 1), lambda i: (i, 0)),
            pl.BlockSpec((R2, 1), lambda i: (i, 0)),
            pl.BlockSpec((1, 1), lambda i: (0, 0)),
            pl.BlockSpec((R1, OUT_PRE), lambda i: (i, 0)),
            pl.BlockSpec((R2, OUT_PRE), lambda i: (i, 0)),
        ],
        out_specs=(pl.BlockSpec((BB, FUTURE), lambda i: (i, 0)),
                   pl.BlockSpec((BB, P, P), lambda i: (i, 0, 0))),
        out_shape=(jax.ShapeDtypeStruct((B, FUTURE), jnp.float32),
                   jax.ShapeDtypeStruct((B, P, P), jnp.float32)),
    )(S4p, hp4, dinv2, bg2b.reshape(1, 1), x_all, x2b)


def kernel(x_idx_sg1, x_float_sg1, x_idx_sg2, x_float_sg2, edge_index_sg1, edge_index_sg2,
           emb0, emb1, W0, b0, W1, b1, W2, b2,
           Wg1a, bg1a, Wg1b, bg1b, Wg2a, bg2a, Wg2b, bg2b,
           Wk, smoothing, M):
    N1 = x_idx_sg1.shape[0]
    N2 = x_idx_sg2.shape[0]
    E1 = edge_index_sg1.shape[1]
    E2 = edge_index_sg2.shape[1]

    s1 = edge_index_sg1[0]
    d1 = edge_index_sg1[1]
    s2 = edge_index_sg2[0]
    d2 = edge_index_sg2[1]
    z_flat = jnp.zeros((N1,), jnp.float32)
    z_2d = jnp.zeros((N1 // _NS, 32), jnp.float32)

    A, theta, T0, T1 = _prep_tc(emb0, emb1, W0, b0, Wk, M)

    deg1p = _make_deg_kernel(N1, E1)(d1, z_flat)
    deg2p = _make_deg_kernel(N2, E2)(d2, z_flat)

    xidx = jnp.concatenate([x_idx_sg1, x_idx_sg2], axis=0)
    xf = jnp.concatenate([x_float_sg1, x_float_sg2], axis=0)
    i0 = xidx[:, 0]
    i1 = xidx[:, 1]
    G0, G1 = _make_emb_gather_kernel(N1 + N2)(T0, T1, i0, i1)
    x_all = _pre_tc(G0, G1, xf, W0[24:27, :], W1, b1, W2, b2)

    hp1A, hp1B, dinv1 = _hp_tc(x_all, deg1p, Wg1a, N1)
    S1 = _make_row_scatter_kernel(N1, E1, False)(hp1A, hp1B, s1, d1, z_2d)

    hp2p = _mid_tc(S1, hp1A, hp1B, dinv1, bg1a, Wg1b, N1)
    S2 = _make_row_scatter_kernel(N1, E1, True)(hp2p, hp2p, s1, d1, z_2d)

    A_tmp = lax.slice(A, (PAST, 0), (PAST + FUTURE, PAST))
    x2b, dinv2, hp3A, hp3B = _att_tc(
        S2, hp2p, dinv1, bg1b, x_all, A_tmp, theta, smoothing,
        deg2p, Wg2a, N1, N2)

    S3 = _make_row_scatter_kernel(N2, E2, False)(hp3A, hp3B, s2, d2, z_2d)

    hp4 = _mid2_tc(S3, hp3A, hp3B, dinv2, bg2a, Wg2b, N2)
    S4p = _make_scalar_scatter_kernel(N2, E2)(s2, d2, hp4[:, 0], z_flat)

    out, dist = _final_tc(S4p, hp4, dinv2, bg2b, x_all, x2b, N1, N2)
    return out, dist, A
